# baseline (speedup 1.0000x reference)
import functools

import jax
import jax.numpy as jnp
from jax import lax
from jax.experimental import pallas as pl
from jax.experimental.pallas import tpu as pltpu
from jax.experimental.pallas import tpu_sc as plsc

BATCH = 16384
DIM = 64

_INFO = plsc.get_sparse_core_info()
_NC = _INFO.num_cores
_NS = _INFO.num_subcores
_NW = _NC * _NS
_BPW = BATCH // _NW
_CHUNK = 128
_NCH = _BPW // _CHUNK

_mesh = plsc.VectorSubcoreMesh(core_axis_name="c", subcore_axis_name="s")


@functools.partial(
    pl.kernel,
    mesh=_mesh,
    out_type=jax.ShapeDtypeStruct((BATCH, 2 * DIM), jnp.float32),
    scratch_types=[
        pltpu.VMEM((_NCH, _CHUNK), jnp.int32),
        pltpu.VMEM((_BPW, DIM), jnp.float32),
        pltpu.VMEM((_BPW, DIM), jnp.float32),
        pltpu.SemaphoreType.DMA,
        pltpu.SemaphoreType.DMA,
    ],
    compiler_params=pltpu.CompilerParams(use_tc_tiling_on_sc=False),
)
def _gather_concat(ids_hbm, table_hbm, out_hbm, idx_v, rows_v, zero_v, sem_g, sem_z):
    wid = lax.axis_index("s") * _NC + lax.axis_index("c")
    base = wid * _BPW

    pltpu.sync_copy(ids_hbm.at[pl.ds(wid * _NCH, _NCH)], idx_v)

    gcopies = []
    for j in range(_NCH):
        c = pltpu.make_async_copy(
            table_hbm.at[idx_v.at[j]],
            rows_v.at[pl.ds(j * _CHUNK, _CHUNK)],
            sem_g,
        )
        c.start()
        gcopies.append(c)

    zvec = jnp.zeros((16,), jnp.float32)

    def _zrow(i, carry):
        for j in range(DIM // 16):
            zero_v[i, pl.ds(j * 16, 16)] = zvec
        return carry

    lax.fori_loop(0, _BPW, _zrow, 0)
    zcopy = pltpu.make_async_copy(
        zero_v, out_hbm.at[pl.ds(base, _BPW), pl.ds(DIM, DIM)], sem_z
    )
    zcopy.start()

    for c in gcopies:
        c.wait()

    pltpu.sync_copy(rows_v, out_hbm.at[pl.ds(base, _BPW), pl.ds(0, DIM)])
    zcopy.wait()


def kernel(entity_ids, entity_table):
    ids = entity_ids.astype(jnp.int32).reshape(_NW * _NCH, _CHUNK)
    return _gather_concat(ids, entity_table)


# speedup vs baseline: 2.5519x; 2.5519x over previous
import functools

import jax
import jax.numpy as jnp
from jax import lax
from jax.experimental import pallas as pl
from jax.experimental.pallas import tpu as pltpu
from jax.experimental.pallas import tpu_sc as plsc

BATCH = 16384
DIM = 64

_INFO = plsc.get_sparse_core_info()
_NC = _INFO.num_cores
_NS = _INFO.num_subcores
_NW = _NC * _NS
_BPW = BATCH // _NW
_L = 16
_K = 4
_BLK = 128
_NBLK = _BPW // _BLK
_GRP = _BLK // _L

_mesh = plsc.VectorSubcoreMesh(core_axis_name="c", subcore_axis_name="s")


@functools.partial(
    pl.kernel,
    mesh=_mesh,
    out_type=jax.ShapeDtypeStruct((BATCH, 2 * DIM), jnp.float32),
    scratch_types=[
        pltpu.VMEM((_BPW + 2 * _L,), jnp.int32),
        [pltpu.VMEM((DIM, 128), jnp.float32) for _ in range(_K)],
        [pltpu.VMEM((_BLK, 2 * DIM), jnp.float32) for _ in range(2)],
        [pltpu.SemaphoreType.DMA for _ in range(_K)],
        [pltpu.SemaphoreType.DMA for _ in range(2)],
    ],
    compiler_params=pltpu.CompilerParams(needs_layout_passes=False),
)
def _gather_concat(ids_hbm, tt_hbm, out_hbm, ids_v, cols, asms, gsems, osems):
    wid = lax.axis_index("s") * _NC + lax.axis_index("c")
    base = wid * _BPW

    pltpu.sync_copy(ids_hbm.at[pl.ds(base, _BPW)], ids_v.at[pl.ds(0, _BPW)])

    def _fetch(slot, eid):
        coloff = pl.multiple_of((eid >> 7) * 128, 128)
        pltpu.make_async_copy(
            tt_hbm.at[:, pl.ds(coloff, 128)], cols[slot], gsems[slot]
        ).start()

    vec0 = ids_v[pl.ds(0, _L)]
    for k in range(_K):
        _fetch(k, vec0[k])

    zvec = jnp.zeros((_L,), jnp.float32)

    def _zrow(i, carry):
        for b in range(2):
            for q in range(DIM // _L):
                asms[b][i, pl.ds(DIM + q * _L, _L)] = zvec
        return carry

    lax.fori_loop(0, _BLK, _zrow, 0)

    iotas = [lax.iota(jnp.int32, _L) + q * _L for q in range(DIM // _L)]

    for blk in range(_NBLK):
        b = blk % 2
        asm = asms[b]
        if blk >= 2:
            pltpu.make_async_copy(
                asm, out_hbm.at[pl.ds(base + (blk - 2) * _BLK, _BLK)], osems[b]
            ).wait()

        def _group(s, carry, blk=blk, asm=asm):
            g0 = blk * _BLK + s * _L
            vec_c = ids_v[pl.ds(g0, _L)]
            vec_n = ids_v[pl.ds(g0 + _L, _L)]
            for k in range(_L):
                slot = k % _K
                pltpu.make_async_copy(
                    tt_hbm.at[:, pl.ds(0, 128)], cols[slot], gsems[slot]
                ).wait()
                lane = vec_c[k] & 127
                lanev = jnp.full((_L,), lane, dtype=jnp.int32)
                row = s * _L + k
                for q in range(DIM // _L):
                    v = plsc.load_gather(cols[slot], [iotas[q], lanev])
                    asm[row, pl.ds(q * _L, _L)] = v
                nid = vec_c[k + _K] if k < _L - _K else vec_n[k + _K - _L]
                if blk < _NBLK - 1:
                    _fetch(slot, nid)
                else:
                    nxt = g0 + k + _K

                    @pl.when(nxt < _BPW)
                    def _():
                        _fetch(slot, nid)
            return carry

        lax.fori_loop(0, _GRP, _group, 0)
        pltpu.make_async_copy(
            asm, out_hbm.at[pl.ds(base + blk * _BLK, _BLK)], osems[b]
        ).start()

    for blk in (_NBLK - 2, _NBLK - 1):
        b = blk % 2
        pltpu.make_async_copy(
            asms[b], out_hbm.at[pl.ds(base + blk * _BLK, _BLK)], osems[b]
        ).wait()


def kernel(entity_ids, entity_table):
    ids = entity_ids.astype(jnp.int32)
    return _gather_concat(ids, entity_table.T)


# speedup vs baseline: 2.9950x; 1.1736x over previous
import functools

import jax
import jax.numpy as jnp
from jax import lax
from jax.experimental import pallas as pl
from jax.experimental.pallas import tpu as pltpu
from jax.experimental.pallas import tpu_sc as plsc

BATCH = 16384
DIM = 64

_INFO = plsc.get_sparse_core_info()
_NC = _INFO.num_cores
_NS = _INFO.num_subcores
_NW = _NC * _NS
_BPW = BATCH // _NW
_L = 16
_K = 8
_BLK = 128
_NBLK = _BPW // _BLK
_GRP = _BLK // _L

_mesh = plsc.VectorSubcoreMesh(core_axis_name="c", subcore_axis_name="s")


@functools.partial(
    pl.kernel,
    mesh=_mesh,
    out_type=jax.ShapeDtypeStruct((BATCH, 2 * DIM), jnp.float32),
    scratch_types=[
        pltpu.VMEM((_BPW + 2 * _L,), jnp.int32),
        [pltpu.VMEM((DIM, 128), jnp.float32) for _ in range(_K)],
        [pltpu.VMEM((_BLK, 2 * DIM), jnp.float32) for _ in range(2)],
        [pltpu.SemaphoreType.DMA for _ in range(_K)],
        [pltpu.SemaphoreType.DMA for _ in range(2)],
    ],
    compiler_params=pltpu.CompilerParams(needs_layout_passes=False),
)
def _gather_concat(ids_hbm, tt_hbm, out_hbm, ids_v, cols, asms, gsems, osems):
    wid = lax.axis_index("s") * _NC + lax.axis_index("c")
    base = wid * _BPW

    pltpu.sync_copy(ids_hbm.at[pl.ds(base, _BPW)], ids_v.at[pl.ds(0, _BPW)])

    def _fetch(slot, eid):
        coloff = pl.multiple_of((eid >> 7) * 128, 128)
        pltpu.make_async_copy(
            tt_hbm.at[:, pl.ds(coloff, 128)], cols[slot], gsems[slot]
        ).start()

    vec0 = ids_v[pl.ds(0, _L)]
    for k in range(_K):
        _fetch(k, vec0[k])

    zvec = jnp.zeros((_L,), jnp.float32)

    def _zrow(i, carry):
        for b in range(2):
            for q in range(DIM // _L):
                asms[b][i, pl.ds(DIM + q * _L, _L)] = zvec
        return carry

    lax.fori_loop(0, _BLK, _zrow, 0)

    iotas = [lax.iota(jnp.int32, _L) + q * _L for q in range(DIM // _L)]

    for blk in range(_NBLK):
        b = blk % 2
        asm = asms[b]
        if blk >= 2:
            pltpu.make_async_copy(
                asm, out_hbm.at[pl.ds(base + (blk - 2) * _BLK, _BLK)], osems[b]
            ).wait()

        def _group(s, carry, blk=blk, asm=asm):
            g0 = blk * _BLK + s * _L
            vec_c = ids_v[pl.ds(g0, _L)]
            vec_n = ids_v[pl.ds(g0 + _L, _L)]
            for k in range(_L):
                slot = k % _K
                pltpu.make_async_copy(
                    tt_hbm.at[:, pl.ds(0, 128)], cols[slot], gsems[slot]
                ).wait()
                lane = vec_c[k] & 127
                lanev = jnp.full((_L,), lane, dtype=jnp.int32)
                row = s * _L + k
                for q in range(DIM // _L):
                    v = plsc.load_gather(cols[slot], [iotas[q], lanev])
                    asm[row, pl.ds(q * _L, _L)] = v
                nid = vec_c[k + _K] if k < _L - _K else vec_n[k + _K - _L]
                if blk < _NBLK - 1:
                    _fetch(slot, nid)
                else:
                    nxt = g0 + k + _K

                    @pl.when(nxt < _BPW)
                    def _():
                        _fetch(slot, nid)
            return carry

        lax.fori_loop(0, _GRP, _group, 0)
        pltpu.make_async_copy(
            asm, out_hbm.at[pl.ds(base + blk * _BLK, _BLK)], osems[b]
        ).start()

    for blk in (_NBLK - 2, _NBLK - 1):
        b = blk % 2
        pltpu.make_async_copy(
            asms[b], out_hbm.at[pl.ds(base + blk * _BLK, _BLK)], osems[b]
        ).wait()


def kernel(entity_ids, entity_table):
    ids = entity_ids.astype(jnp.int32)
    return _gather_concat(ids, entity_table.T)
